# baseline (speedup 1.0000x reference)
import functools

import jax
import jax.numpy as jnp
from jax import lax
from jax.experimental import pallas as pl
from jax.experimental.pallas import tpu as pltpu
from jax.experimental.pallas import tpu_sc as plsc

BATCH = 16384
EMBED_DIM = 32
NUM_ROWS = 1000000
NUM_WORKERS = 32
B_PER_W = BATCH // NUM_WORKERS
LANES = 16

FULL_BLOCKS = 7812
TAIL_START = FULL_BLOCKS * 128
CH_BLOCKS = 8
CH_COLS = CH_BLOCKS * 128
K1_TRIPS = 31
P_ROWS = NUM_ROWS // 4
IDX_CHUNK = 128


@functools.lru_cache(maxsize=1)
def _make_kernels():
  mesh = plsc.VectorSubcoreMesh(core_axis_name="c", subcore_axis_name="s")

  @functools.partial(
      pl.kernel,
      mesh=mesh,
      compiler_params=pltpu.CompilerParams(needs_layout_passes=False),
      out_type=jax.ShapeDtypeStruct((P_ROWS, 128), jnp.float32),
      scratch_types=[
          pltpu.VMEM((4, 8, CH_COLS), jnp.float32),
          pltpu.VMEM((CH_COLS // 4, 128), jnp.float32),
          pltpu.SemaphoreType.DMA,
      ],
  )
  def k1(table3, p_out, buf, obuf, sem):
    wid = lax.axis_index("s") * 2 + lax.axis_index("c")
    b0 = wid * 244 + jnp.minimum(wid, 4)
    nblk = jnp.where(wid < 4, 245, 244)

    lanes = lax.iota(jnp.int32, LANES)
    row_off = lanes >> 2
    col_off = (lanes & 3) * 32

    def trip(c, carry):
      blk = jnp.minimum(b0 + c * CH_BLOCKS, b0 + nblk - CH_BLOCKS)
      col0 = pl.multiple_of(blk * 128, 128)
      for i in range(4):
        pltpu.async_copy(table3.at[i, :, pl.ds(col0, CH_COLS)],
                         buf.at[i], sem)
      for i in range(4):
        pltpu.make_async_copy(table3.at[i, :, pl.ds(0, CH_COLS)],
                              buf.at[i], sem).wait()
      for d in range(EMBED_DIM):
        i, k = d // 8, d % 8
        ksel = jnp.full((LANES,), k, jnp.int32)

        def g_body(g, carry2):
          cols = g * LANES + lanes
          v = plsc.load_gather(buf.at[i], [ksel, cols])
          rows = g * 4 + row_off
          plsc.store_scatter(obuf, [rows, col_off + d], v)
          return carry2

        lax.fori_loop(0, CH_COLS // LANES, g_body, 0)
      pltpu.sync_copy(obuf, p_out.at[pl.ds(blk * 32, CH_COLS // 4)])
      return carry

    lax.fori_loop(0, K1_TRIPS, trip, 0)

  @functools.partial(
      pl.kernel,
      mesh=mesh,
      compiler_params=pltpu.CompilerParams(needs_layout_passes=False),
      out_type=jax.ShapeDtypeStruct((BATCH,), jnp.float32),
      scratch_types=[
          pltpu.VMEM((B_PER_W,), jnp.int32),
          pltpu.VMEM((B_PER_W,), jnp.int32),
          pltpu.VMEM((B_PER_W,), jnp.int32),
          pltpu.VMEM((IDX_CHUNK, 128), jnp.float32),
          pltpu.VMEM((IDX_CHUNK, 128), jnp.float32),
          pltpu.VMEM((2048,), jnp.float32),
          pltpu.VMEM((B_PER_W,), jnp.float32),
          pltpu.SemaphoreType.DMA,
      ],
  )
  def k2(central_hbm, context_hbm, p_hbm, tail_hbm, out_hbm,
         cidx_v, xidx_v, slot_v, crows_v, xrows_v, tail_v, out_v, sem):
    wid = lax.axis_index("s") * 2 + lax.axis_index("c")
    base = wid * B_PER_W
    lanes = lax.iota(jnp.int32, LANES)

    pltpu.sync_copy(central_hbm.at[pl.ds(base, B_PER_W)], cidx_v)
    pltpu.sync_copy(context_hbm.at[pl.ds(base, B_PER_W)], xidx_v)
    pltpu.sync_copy(tail_hbm, tail_v)

    def chunk_body(c, carry):
      coff = c * IDX_CHUNK
      def slot_body(v, carry2):
        off = pl.ds(coff + v * LANES, LANES)
        cidx = cidx_v[off]
        slot_v[pl.ds(v * LANES, LANES)] = cidx >> 2
        xidx = xidx_v[off]
        slot_v[pl.ds(IDX_CHUNK + v * LANES, LANES)] = xidx >> 2
        return carry2

      lax.fori_loop(0, IDX_CHUNK // LANES, slot_body, 0)
      cp1 = pltpu.async_copy(p_hbm.at[slot_v.at[pl.ds(0, IDX_CHUNK)]],
                             crows_v, sem)
      cp2 = pltpu.async_copy(p_hbm.at[slot_v.at[pl.ds(IDX_CHUNK, IDX_CHUNK)]],
                             xrows_v, sem)
      cp1.wait()
      cp2.wait()

      def dot_body(b, carry2):
        off = pl.ds(coff + b * LANES, LANES)
        rc = cidx_v[off]
        rx = xidx_v[off]
        rowsel = b * LANES + lanes
        qc = (rc & 3) * 32
        qx = (rx & 3) * 32
        cmask = rc >= TAIL_START
        xmask = rx >= TAIL_START
        ct = jnp.minimum(jnp.maximum(rc - TAIL_START, 0), 63)
        xt = jnp.minimum(jnp.maximum(rx - TAIL_START, 0), 63)
        acc = jnp.zeros((LANES,), jnp.float32)
        for d in range(EMBED_DIM):
          cv = plsc.load_gather(crows_v, [rowsel, qc + d])
          xv = plsc.load_gather(xrows_v, [rowsel, qx + d])
          tc = plsc.load_gather(tail_v, [d * 64 + ct])
          tx = plsc.load_gather(tail_v, [d * 64 + xt])
          cv = jnp.where(cmask, tc, cv)
          xv = jnp.where(xmask, tx, xv)
          acc = acc + cv * xv
        out_v[pl.ds(coff + b * LANES, LANES)] = acc
        return carry2

      lax.fori_loop(0, IDX_CHUNK // LANES, dot_body, 0)
      return carry

    lax.fori_loop(0, B_PER_W // IDX_CHUNK, chunk_body, 0)
    pltpu.sync_copy(out_v, out_hbm.at[pl.ds(base, B_PER_W)])

  return k1, k2


def kernel(central_idx, context_idx, embeddings):
  central_idx = central_idx.astype(jnp.int32)
  context_idx = context_idx.astype(jnp.int32)
  k1, k2 = _make_kernels()
  table3 = embeddings.T.reshape(4, 8, NUM_ROWS)
  packed = k1(table3)
  tail_flat = embeddings[TAIL_START:].T.reshape(-1)
  return k2(central_idx, context_idx, packed, tail_flat)


# speedup vs baseline: 1.5132x; 1.5132x over previous
import functools

import jax
import jax.numpy as jnp
from jax import lax
from jax.experimental import pallas as pl
from jax.experimental.pallas import tpu as pltpu
from jax.experimental.pallas import tpu_sc as plsc

BATCH = 16384
EMBED_DIM = 32
NUM_WORKERS = 32
B_PER_W = BATCH // NUM_WORKERS
IDX_CHUNK = 128
N_CHUNKS = B_PER_W // IDX_CHUNK
LANES = 16


@functools.lru_cache(maxsize=1)
def _make_sc_kernel():
  mesh = plsc.VectorSubcoreMesh(core_axis_name="c", subcore_axis_name="s")

  @functools.partial(
      pl.kernel,
      mesh=mesh,
      compiler_params=pltpu.CompilerParams(
          needs_layout_passes=False, use_tc_tiling_on_sc=False),
      out_type=jax.ShapeDtypeStruct((BATCH,), jnp.float32),
      scratch_types=[
          pltpu.VMEM((N_CHUNKS, IDX_CHUNK), jnp.int32),
          pltpu.VMEM((N_CHUNKS, IDX_CHUNK), jnp.int32),
          pltpu.VMEM((B_PER_W, EMBED_DIM), jnp.float32),
          pltpu.VMEM((B_PER_W, EMBED_DIM), jnp.float32),
          pltpu.VMEM((B_PER_W,), jnp.float32),
          pltpu.SemaphoreType.DMA,
      ],
  )
  def sc_kernel(central_hbm, context_hbm, table_hbm, out_hbm,
                cidx_v, xidx_v, crows_v, xrows_v, out_v, sem):
    wid = lax.axis_index("s") * 2 + lax.axis_index("c")
    base = wid * B_PER_W

    for c in range(N_CHUNKS):
      off = base + c * IDX_CHUNK
      pltpu.sync_copy(central_hbm.at[pl.ds(off, IDX_CHUNK)], cidx_v.at[c])
      pltpu.sync_copy(context_hbm.at[pl.ds(off, IDX_CHUNK)], xidx_v.at[c])

    copies = []
    for c in range(N_CHUNKS):
      dst = pl.ds(c * IDX_CHUNK, IDX_CHUNK)
      copies.append(pltpu.async_copy(table_hbm.at[cidx_v.at[c]],
                                     crows_v.at[dst], sem))
      copies.append(pltpu.async_copy(table_hbm.at[xidx_v.at[c]],
                                     xrows_v.at[dst], sem))
    for cp in copies:
      cp.wait()

    lanes = lax.iota(jnp.int32, LANES)

    def block_body(b, carry):
      rbase = b * LANES
      acc = jnp.zeros((LANES,), jnp.float32)
      for u in range(LANES):
        r = rbase + u
        c0 = crows_v[r, pl.ds(0, LANES)]
        c1 = crows_v[r, pl.ds(LANES, LANES)]
        x0 = xrows_v[r, pl.ds(0, LANES)]
        x1 = xrows_v[r, pl.ds(LANES, LANES)]
        s = jnp.sum(c0 * x0 + c1 * x1)
        acc = jnp.where(lanes == u, s, acc)
      out_v[pl.ds(rbase, LANES)] = acc
      return carry

    lax.fori_loop(0, B_PER_W // LANES, block_body, 0)

    pltpu.sync_copy(out_v, out_hbm.at[pl.ds(base, B_PER_W)])

  return sc_kernel


def kernel(central_idx, context_idx, embeddings):
  central_idx = central_idx.astype(jnp.int32)
  context_idx = context_idx.astype(jnp.int32)
  return _make_sc_kernel()(central_idx, context_idx, embeddings)
